# baseline (speedup 1.0000x reference)
import functools

import jax
import jax.numpy as jnp
from jax import lax
from jax.experimental import pallas as pl
from jax.experimental.pallas import tpu as pltpu
from jax.experimental.pallas import tpu_sc as plsc

D = 32
GRP = 128
NW = 32


def _sc_gather(idxT, table):
    F, B = idxT.shape
    BW = B // NW
    NG = BW // GRP
    mesh = plsc.VectorSubcoreMesh(core_axis_name="c", subcore_axis_name="s")

    @functools.partial(
        pl.kernel,
        out_type=jax.ShapeDtypeStruct((B, F, D), jnp.float32),
        mesh=mesh,
        compiler_params=pltpu.CompilerParams(use_tc_tiling_on_sc=False),
        scratch_types=[
            pltpu.VMEM((F, BW), jnp.int32),
            pltpu.VMEM((2, BW, D), jnp.float32),
            pltpu.SemaphoreType.DMA,
            pltpu.SemaphoreType.DMA,
        ],
    )
    def k(idxT_hbm, table_hbm, out_hbm, idx_v, rows_v, sem0, sem1):
        sems = (sem0, sem1)
        wid = lax.axis_index("s") * 2 + lax.axis_index("c")
        b0 = wid * BW
        pltpu.sync_copy(idxT_hbm.at[:, pl.ds(b0, BW)], idx_v)

        def fire(f, buf):
            for g in range(NG):
                pltpu.async_copy(
                    table_hbm.at[idx_v.at[f, pl.ds(g * GRP, GRP)]],
                    rows_v.at[buf, pl.ds(g * GRP, GRP)],
                    sems[buf],
                )

        def drain(buf):
            for g in range(NG):
                pltpu.make_async_copy(
                    table_hbm.at[pl.ds(0, GRP)],
                    rows_v.at[buf, pl.ds(g * GRP, GRP)],
                    sems[buf],
                ).wait()

        def store(f, buf):
            pltpu.sync_copy(
                rows_v.at[buf],
                out_hbm.at[pl.ds(b0, BW), f, :],
            )

        fire(0, 0)

        def body(i, carry):
            f0 = 2 * i
            fire(f0 + 1, 1)
            drain(0)
            store(f0, 0)
            fire(f0 + 2, 0)
            drain(1)
            store(f0 + 1, 1)
            return carry

        lax.fori_loop(0, (F - 2) // 2, body, 0)
        fire(F - 1, 1)
        drain(0)
        store(F - 2, 0)
        drain(1)
        store(F - 1, 1)

    return k(idxT, table)


def kernel(indices, values_weight):
    idxT = indices.astype(jnp.int32).T
    return _sc_gather(idxT, values_weight)
